# baseline (speedup 1.0000x reference)
import functools

import jax
import jax.numpy as jnp
from jax import lax
from jax.experimental import pallas as pl
from jax.experimental.pallas import tpu as pltpu
from jax.experimental.pallas import tpu_sc as plsc

_NLE = 4096
_E = 131072
_NSUB = 16
_NCORE = 2
_EPT = _E // _NSUB
_PASS_ROWS = 128
_ROWS_PER_SC = _NLE // _NCORE
_NPASS = _ROWS_PER_SC // _PASS_ROWS
_ZROWS = _PASS_ROWS // _NSUB


_NBKT = _NLE // _PASS_ROWS
_BWIN = 512
_ZWORDS = _ZROWS * _NLE
_SHARD = _PASS_ROWS * _NLE
_SHIFT = _PASS_ROWS.bit_length() - 1


_DEGBASE = _SHARD + 128


def _sc_build_counts_body(dst_hbm, src_hbm, c_hbm, deg_hbm, dstv, srcv, binv,
                          idxw, valw, zbuf, shard, sem):
    cid = lax.axis_index("c")
    sid = lax.axis_index("s")
    pltpu.sync_copy(dst_hbm.at[pl.ds(sid * _EPT, _EPT)], dstv)
    pltpu.sync_copy(src_hbm.at[pl.ds(sid * _EPT, _EPT)], srcv)

    def _zero(i, carry):
        zbuf[pl.ds(i * 16, 16)] = jnp.zeros((16,), jnp.float32)
        return carry

    lax.fori_loop(0, _ZWORDS // 16, _zero, 0)

    for c in range(8):
        valw[pl.ds(c * 16, 16)] = jnp.full((16,), 1.0, jnp.float32)

    def _fill(i, carry):
        binv[pl.ds(i * 16, 16)] = jnp.full((16,), -1, jnp.int32)
        return carry

    lax.fori_loop(0, _NBKT * _BWIN // 16, _fill, 0)

    lane = lax.iota(jnp.int32, 16)

    def _bin(i, carry):
        cnt_lo, cnt_hi = carry
        d = dstv[pl.ds(i * 16, 16)]
        s = srcv[pl.ds(i * 16, 16)]
        b = lax.shift_right_logical(d, _SHIFT)
        gidx = d * _NLE + s
        pos = jnp.zeros((16,), jnp.int32)
        for j in range(_NBKT):
            eq = b == j
            pc = plsc.cumsum(jnp.where(eq, 1, 0))
            bank = cnt_lo if j < 16 else cnt_hi
            base = jnp.sum(jnp.where(lane == (j % 16), bank, 0))
            pos = jnp.where(eq, base + pc - 1, pos)
            add = plsc.all_reduce_population_count(eq)
            bank = jnp.where(lane == (j % 16), bank + add, bank)
            if j < 16:
                cnt_lo = bank
            else:
                cnt_hi = bank
        plsc.store_scatter(binv, [b * _BWIN + pos], gidx)
        return cnt_lo, cnt_hi

    zeros16 = jnp.zeros((16,), jnp.int32)
    lax.fori_loop(0, _EPT // 16, _bin, (zeros16, zeros16))

    @pl.when(sid == 0)
    def _():
        pltpu.sync_copy(zbuf.at[pl.ds(0, _NLE)],
                        shard.at[pl.ds(_DEGBASE, _NLE)])

    plsc.subcore_barrier()
    for q in range(_EPT // _BWIN):

        def _mkd(j, carry):
            dv = dstv[pl.ds(q * _BWIN + j * 16, 16)]
            sv = srcv[pl.ds(q * _BWIN + j * 16, 16)]
            idxw[j // 8, pl.ds((j % 8) * 16, 16)] = jnp.where(
                cid == 0, dv, sv) + _DEGBASE
            return carry

        lax.fori_loop(0, _BWIN // 16, _mkd, 0)
        dcopies = [
            pltpu.async_copy(valw, shard.at[idxw.at[j]], sem, add=True)
            for j in range(_BWIN // 128)
        ]
        for c in dcopies:
            c.wait()
    plsc.subcore_barrier()

    @pl.when(sid == 0)
    def _():
        pltpu.sync_copy(shard.at[pl.ds(_DEGBASE, _NLE)],
                        deg_hbm.at[pl.ds(cid * _NLE, _NLE)])

    for p in range(_NPASS):
        bkt = cid * _NPASS + p
        base_flat = bkt * _SHARD
        pltpu.sync_copy(zbuf, shard.at[pl.ds(sid * _ZWORDS, _ZWORDS)])
        plsc.subcore_barrier()

        def _mk(j, carry):
            g = binv[pl.ds(bkt * _BWIN + j * 16, 16)]
            live = g >= 0
            idxw[j // 8, pl.ds((j % 8) * 16, 16)] = jnp.where(
                live, g - base_flat, _SHARD + lane)
            return carry

        lax.fori_loop(0, _BWIN // 16, _mk, 0)
        copies = [
            pltpu.async_copy(valw, shard.at[idxw.at[j]], sem, add=True)
            for j in range(_BWIN // 128)
        ]
        for c in copies:
            c.wait()
        plsc.subcore_barrier()
        pltpu.sync_copy(
            shard.at[pl.ds(sid * _ZWORDS, _ZWORDS)],
            c_hbm.at[pl.ds(base_flat + sid * _ZWORDS, _ZWORDS)])
        plsc.subcore_barrier()


@functools.cache
def _sc_build_counts():
    return pl.kernel(
        _sc_build_counts_body,
        out_type=[jax.ShapeDtypeStruct((_NLE * _NLE,), jnp.float32),
                  jax.ShapeDtypeStruct((2 * _NLE,), jnp.float32)],
        mesh=plsc.VectorSubcoreMesh(core_axis_name="c", subcore_axis_name="s",
                                    num_cores=_NCORE, num_subcores=_NSUB),
        scratch_types=[
            pltpu.VMEM((_EPT,), jnp.int32),
            pltpu.VMEM((_EPT,), jnp.int32),
            pltpu.VMEM((_NBKT * _BWIN,), jnp.int32),
            pltpu.VMEM((_BWIN // 128, 128), jnp.int32),
            pltpu.VMEM((128,), jnp.float32),
            pltpu.VMEM((_ZWORDS,), jnp.float32),
            pltpu.VMEM_SHARED((_DEGBASE + _NLE,), jnp.float32),
            pltpu.SemaphoreType.DMA,
        ],
        compiler_params=pltpu.CompilerParams(needs_layout_passes=False),
    )


def _degrees(C):
    bm = 256
    ng = _NLE // bm

    def kern(c_ref, g_ref, f_ref):
        i = pl.program_id(0)
        blk = c_ref[...].astype(jnp.float32)
        g_ref[...] = lax.rsqrt(jnp.maximum(jnp.sum(blk, axis=1), 1.0))
        cs = jnp.sum(blk, axis=0)

        @pl.when(i == 0)
        def _():
            f_ref[...] = cs

        @pl.when(i != 0)
        def _():
            f_ref[...] = f_ref[...] + cs

        @pl.when(i == ng - 1)
        def _():
            f_ref[...] = lax.rsqrt(jnp.maximum(f_ref[...], 1.0))

    g, f = pl.pallas_call(
        kern,
        grid=(ng,),
        in_specs=[pl.BlockSpec((bm, _NLE), lambda i: (i, 0))],
        out_specs=[pl.BlockSpec((bm,), lambda i: (i,)),
                   pl.BlockSpec((_NLE,), lambda i: (0,))],
        out_shape=[jax.ShapeDtypeStruct((_NLE,), jnp.float32),
                   jax.ShapeDtypeStruct((_NLE,), jnp.float32)],
        compiler_params=pltpu.CompilerParams(
            dimension_semantics=("arbitrary",)),
    )(C)
    return f, g


def _matmul(A, B, scale=None, bias=None, act=None, bm=512,
            low_prec=False, scale_rsqrt=False):
    M, K = A.shape
    Nn = B.shape[1]
    bm = min(bm, M)

    operands = [A, B]
    in_specs = [
        pl.BlockSpec((bm, K), lambda i: (i, 0)),
        pl.BlockSpec((K, Nn), lambda i: (0, 0)),
    ]
    if scale is not None:
        operands.append(scale)
        in_specs.append(pl.BlockSpec((bm,), lambda i: (i,)))
    if bias is not None:
        operands.append(bias)
        in_specs.append(pl.BlockSpec((Nn,), lambda i: (0,)))

    def kern(*refs):
        a, b, o = refs[0], refs[1], refs[-1]
        extra = refs[2:-1]
        av, bv = a[...], b[...]
        if low_prec:
            av = av.astype(jnp.bfloat16)
            bv = bv.astype(jnp.bfloat16)
        v = jnp.dot(av, bv, preferred_element_type=jnp.float32)
        j = 0
        if scale is not None:
            sv = extra[j][...]
            if scale_rsqrt:
                sv = lax.rsqrt(jnp.maximum(sv, 1.0))
            v = v * sv[:, None]
            j += 1
        if bias is not None:
            v = v + extra[j][...][None, :]
        if act is not None:
            v = act(v)
        o[...] = v

    return pl.pallas_call(
        kern,
        grid=(M // bm,),
        in_specs=in_specs,
        out_specs=pl.BlockSpec((bm, Nn), lambda i: (i, 0)),
        out_shape=jax.ShapeDtypeStruct((M, Nn), jnp.float32),
        compiler_params=pltpu.CompilerParams(
            dimension_semantics=("arbitrary",)),
    )(*operands)


def _relu(v):
    return jnp.maximum(v, 0.0)


def _sigmoid(v):
    return 1.0 / (1.0 + jnp.exp(-v))


def kernel(x, Pv, PvT, edge_index, W1, b1, W2, b2, Wout, bout):
    src = edge_index[0].astype(jnp.int32)
    dst = edge_index[1].astype(jnp.int32)
    Cflat, deg = _sc_build_counts()(dst, src)
    C = Cflat.reshape(_NLE, _NLE)
    g = deg[:_NLE]
    f = deg[_NLE:]

    h = _matmul(x, W1, bias=b1)
    h = _matmul(Pv, h, scale=f, scale_rsqrt=True, low_prec=True, bm=256)
    h = _matmul(C, h, scale=g, scale_rsqrt=True, low_prec=True, bm=256)
    h = _matmul(PvT, h, act=_relu, low_prec=True, bm=256)
    h = _matmul(h, W2, bias=b2)
    h = _matmul(Pv, h, scale=f, scale_rsqrt=True, low_prec=True, bm=256)
    h = _matmul(C, h, scale=g, scale_rsqrt=True, low_prec=True, bm=256)
    h = _matmul(PvT, h, low_prec=True, bm=256)
    Wp = jnp.pad(Wout, ((0, 0), (0, 127)))
    bp = jnp.pad(bout, (0, 127))
    out = _matmul(h, Wp, bias=bp, act=_sigmoid)
    return out[:, :1]


# speedup vs baseline: 1.0441x; 1.0441x over previous
import functools

import jax
import jax.numpy as jnp
from jax import lax
from jax.experimental import pallas as pl
from jax.experimental.pallas import tpu as pltpu
from jax.experimental.pallas import tpu_sc as plsc

_NLE = 4096
_E = 131072
_NSUB = 16
_NCORE = 2
_EPT = _E // _NSUB
_PASS_ROWS = 128
_ROWS_PER_SC = _NLE // _NCORE
_NPASS = _ROWS_PER_SC // _PASS_ROWS
_ZROWS = _PASS_ROWS // _NSUB


_NBKT = _NLE // _PASS_ROWS
_BWIN = 512
_ZWORDS = _ZROWS * _NLE
_SHARD = _PASS_ROWS * _NLE
_SHIFT = _PASS_ROWS.bit_length() - 1


_DEGBASE = _SHARD + 128


def _sc_build_counts_body(dst_hbm, src_hbm, c_hbm, deg_hbm, dstv, srcv, binv,
                          idxw, valw, zbuf, shard, sem):
    cid = lax.axis_index("c")
    sid = lax.axis_index("s")
    pltpu.sync_copy(dst_hbm.at[pl.ds(sid * _EPT, _EPT)], dstv)
    pltpu.sync_copy(src_hbm.at[pl.ds(sid * _EPT, _EPT)], srcv)

    def _zero(i, carry):
        zbuf[pl.ds(i * 16, 16)] = jnp.zeros((16,), jnp.float32)
        return carry

    lax.fori_loop(0, _ZWORDS // 16, _zero, 0)

    for c in range(8):
        valw[pl.ds(c * 16, 16)] = jnp.full((16,), 1.0, jnp.float32)

    def _fill(i, carry):
        binv[pl.ds(i * 16, 16)] = jnp.full((16,), -1, jnp.int32)
        return carry

    lax.fori_loop(0, _NBKT * _BWIN // 16, _fill, 0)

    lane = lax.iota(jnp.int32, 16)

    def _bin(i, carry):
        cnt_lo, cnt_hi = carry
        d = dstv[pl.ds(i * 16, 16)]
        s = srcv[pl.ds(i * 16, 16)]
        b = lax.shift_right_logical(d, _SHIFT)
        gidx = d * _NLE + s
        pos = jnp.zeros((16,), jnp.int32)
        for j in range(_NBKT):
            eq = b == j
            pc = plsc.cumsum(jnp.where(eq, 1, 0))
            bank = cnt_lo if j < 16 else cnt_hi
            base = jnp.sum(jnp.where(lane == (j % 16), bank, 0))
            pos = jnp.where(eq, base + pc - 1, pos)
            add = plsc.all_reduce_population_count(eq)
            bank = jnp.where(lane == (j % 16), bank + add, bank)
            if j < 16:
                cnt_lo = bank
            else:
                cnt_hi = bank
        plsc.store_scatter(binv, [b * _BWIN + pos], gidx)
        return cnt_lo, cnt_hi

    zeros16 = jnp.zeros((16,), jnp.int32)
    lax.fori_loop(0, _EPT // 16, _bin, (zeros16, zeros16))

    @pl.when(sid == 0)
    def _():
        pltpu.sync_copy(zbuf.at[pl.ds(0, _NLE)],
                        shard.at[pl.ds(_DEGBASE, _NLE)])

    plsc.subcore_barrier()
    for q in range(_EPT // _BWIN):

        def _mkd(j, carry):
            dv = dstv[pl.ds(q * _BWIN + j * 16, 16)]
            sv = srcv[pl.ds(q * _BWIN + j * 16, 16)]
            idxw[j // 8, pl.ds((j % 8) * 16, 16)] = jnp.where(
                cid == 0, dv, sv) + _DEGBASE
            return carry

        lax.fori_loop(0, _BWIN // 16, _mkd, 0)
        dcopies = [
            pltpu.async_copy(valw, shard.at[idxw.at[j]], sem, add=True)
            for j in range(_BWIN // 128)
        ]
        for c in dcopies:
            c.wait()
    plsc.subcore_barrier()

    @pl.when(sid == 0)
    def _():
        pltpu.sync_copy(shard.at[pl.ds(_DEGBASE, _NLE)],
                        deg_hbm.at[pl.ds(cid * _NLE, _NLE)])

    for p in range(_NPASS):
        bkt = cid * _NPASS + p
        base_flat = bkt * _SHARD
        pltpu.sync_copy(zbuf, shard.at[pl.ds(sid * _ZWORDS, _ZWORDS)])
        plsc.subcore_barrier()

        def _mk(j, carry):
            g = binv[pl.ds(bkt * _BWIN + j * 16, 16)]
            live = g >= 0
            idxw[j // 8, pl.ds((j % 8) * 16, 16)] = jnp.where(
                live, g - base_flat, _SHARD + lane)
            return carry

        lax.fori_loop(0, _BWIN // 16, _mk, 0)
        copies = [
            pltpu.async_copy(valw, shard.at[idxw.at[j]], sem, add=True)
            for j in range(_BWIN // 128)
        ]
        for c in copies:
            c.wait()
        plsc.subcore_barrier()
        pltpu.sync_copy(
            shard.at[pl.ds(sid * _ZWORDS, _ZWORDS)],
            c_hbm.at[pl.ds(base_flat + sid * _ZWORDS, _ZWORDS)])
        plsc.subcore_barrier()


@functools.cache
def _sc_build_counts():
    return pl.kernel(
        _sc_build_counts_body,
        out_type=[jax.ShapeDtypeStruct((_NLE * _NLE,), jnp.float32),
                  jax.ShapeDtypeStruct((2 * _NLE,), jnp.float32)],
        mesh=plsc.VectorSubcoreMesh(core_axis_name="c", subcore_axis_name="s",
                                    num_cores=_NCORE, num_subcores=_NSUB),
        scratch_types=[
            pltpu.VMEM((_EPT,), jnp.int32),
            pltpu.VMEM((_EPT,), jnp.int32),
            pltpu.VMEM((_NBKT * _BWIN,), jnp.int32),
            pltpu.VMEM((_BWIN // 128, 128), jnp.int32),
            pltpu.VMEM((128,), jnp.float32),
            pltpu.VMEM((_ZWORDS,), jnp.float32),
            pltpu.VMEM_SHARED((_DEGBASE + _NLE,), jnp.float32),
            pltpu.SemaphoreType.DMA,
        ],
        compiler_params=pltpu.CompilerParams(needs_layout_passes=False),
    )


def _matmul(A, B, scale=None, bias=None, act=None, bm=512,
            low_prec=False, scale_rsqrt=False):
    M, K = A.shape
    Nn = B.shape[1]
    bm = min(bm, M)

    operands = [A, B]
    in_specs = [
        pl.BlockSpec((bm, K), lambda i: (i, 0)),
        pl.BlockSpec((K, Nn), lambda i: (0, 0)),
    ]
    if scale is not None:
        operands.append(scale)
        in_specs.append(pl.BlockSpec((bm,), lambda i: (i,)))
    if bias is not None:
        operands.append(bias)
        in_specs.append(pl.BlockSpec((Nn,), lambda i: (0,)))

    def kern(*refs):
        a, b, o = refs[0], refs[1], refs[-1]
        extra = refs[2:-1]
        av, bv = a[...], b[...]
        if low_prec:
            av = av.astype(jnp.bfloat16)
            bv = bv.astype(jnp.bfloat16)
        v = jnp.dot(av, bv, preferred_element_type=jnp.float32)
        j = 0
        if scale is not None:
            sv = extra[j][...]
            if scale_rsqrt:
                sv = lax.rsqrt(jnp.maximum(sv, 1.0))
            v = v * sv[:, None]
            j += 1
        if bias is not None:
            v = v + extra[j][...][None, :]
        if act is not None:
            v = act(v)
        o[...] = v

    return pl.pallas_call(
        kern,
        grid=(M // bm,),
        in_specs=in_specs,
        out_specs=pl.BlockSpec((bm, Nn), lambda i: (i, 0)),
        out_shape=jax.ShapeDtypeStruct((M, Nn), jnp.float32),
        compiler_params=pltpu.CompilerParams(
            dimension_semantics=("arbitrary",)),
    )(*operands)


def _relu(v):
    return jnp.maximum(v, 0.0)


def _sigmoid(v):
    return 1.0 / (1.0 + jnp.exp(-v))


def kernel(x, Pv, PvT, edge_index, W1, b1, W2, b2, Wout, bout):
    src = edge_index[0].astype(jnp.int32)
    dst = edge_index[1].astype(jnp.int32)
    Cflat, deg = _sc_build_counts()(dst, src)
    C = Cflat.reshape(_NLE, _NLE)
    g = deg[:_NLE]
    f = deg[_NLE:]

    h = _matmul(x, W1, bias=b1)
    h = _matmul(Pv, h, scale=f, scale_rsqrt=True, low_prec=True, bm=512)
    h = _matmul(C, h, scale=g, scale_rsqrt=True, low_prec=True, bm=512)
    h = _matmul(PvT, h, act=_relu, low_prec=True, bm=512)
    h = _matmul(h, W2, bias=b2)
    h = _matmul(Pv, h, scale=f, scale_rsqrt=True, low_prec=True, bm=512)
    h = _matmul(C, h, scale=g, scale_rsqrt=True, low_prec=True, bm=512)
    h = _matmul(PvT, h, low_prec=True, bm=512)
    Wp = jnp.pad(Wout, ((0, 0), (0, 127)))
    bp = jnp.pad(bout, (0, 127))
    out = _matmul(h, Wp, bias=bp, act=_sigmoid)
    return out[:, :1]
